# baseline (speedup 1.0000x reference)
import functools

import jax
import jax.numpy as jnp
import numpy as np
from jax.experimental import pallas as pl
from jax.experimental.pallas import tpu as pltpu

try:
    from jax.experimental.shard_map import shard_map as _shard_map
except ImportError:
    from jax import shard_map as _shard_map

from jax.sharding import Mesh, PartitionSpec as P

_Q = 1024
_D = 16
_BN = 4096


def _knn_step(x_ref, x2_ref, cb_ref, fcols_ref, oval_ref, oidx_ref,
              minval, minidx, *, n_total):
    i = pl.program_id(0)
    nsteps = pl.num_programs(0)

    x = x_ref[...]
    x2 = x2_ref[...]
    cb = cb_ref[...]

    row = jax.lax.broadcasted_iota(jnp.int32, (_BN, 1), 0)
    valid = (i * _BN + row) < n_total
    cb = jnp.where(valid, cb, 0.0)

    xsq = jnp.sum(x * x, axis=1, keepdims=True)
    csq = jnp.sum(cb * cb, axis=1, keepdims=True)
    csq = jnp.where(valid, csq, jnp.float32(3e38))

    mm2 = jax.lax.dot_general(
        x2, cb, (((1,), (1,)), ((), ())),
        preferred_element_type=jnp.float32)
    d = xsq - mm2 + csq.T

    gcols = fcols_ref[...] + (i * _BN).astype(jnp.float32)

    pairs = [(d[:, k * 128:(k + 1) * 128], gcols[:, k * 128:(k + 1) * 128])
             for k in range(_BN // 128)]
    while len(pairs) > 1:
        nxt = []
        for k in range(0, len(pairs) - 1, 2):
            (av, ai), (bv, bi) = pairs[k], pairs[k + 1]
            take_b = bv < av
            nxt.append((jnp.minimum(av, bv), jnp.where(take_b, bi, ai)))
        if len(pairs) % 2:
            nxt.append(pairs[-1])
        pairs = nxt
    lval, lid = pairs[0]

    @pl.when(i == 0)
    def _():
        minval[...] = lval
        minidx[...] = lid

    @pl.when(i > 0)
    def _():
        take = lval < minval[...]
        minval[...] = jnp.minimum(minval[...], lval)
        minidx[...] = jnp.where(take, lid, minidx[...])

    @pl.when(i == nsteps - 1)
    def _():
        accv = minval[...]
        acci = minidx[...]
        m = jnp.min(accv, axis=1, keepdims=True)
        idxf = jnp.min(
            jnp.where(accv == m, acci, jnp.float32(3e38)),
            axis=1, keepdims=True)
        oval_ref[...] = m
        oidx_ref[...] = idxf.astype(jnp.int32)


def _knn_pallas(x, cb, n_total):
    nsteps = (n_total + _BN - 1) // _BN
    return pl.pallas_call(
        functools.partial(_knn_step, n_total=n_total),
        grid=(nsteps,),
        in_specs=[
            pl.BlockSpec((_Q, _D), lambda i: (0, 0)),
            pl.BlockSpec((_Q, _D), lambda i: (0, 0)),
            pl.BlockSpec((_BN, _D), lambda i: (i, 0)),
            pl.BlockSpec((1, _BN), lambda i: (0, 0)),
        ],
        out_specs=[
            pl.BlockSpec((_Q, 1), lambda i: (0, 0)),
            pl.BlockSpec((_Q, 1), lambda i: (0, 0)),
        ],
        out_shape=[
            jax.ShapeDtypeStruct((_Q, 1), jnp.float32),
            jax.ShapeDtypeStruct((_Q, 1), jnp.int32),
        ],
        scratch_shapes=[
            pltpu.VMEM((_Q, 128), jnp.float32),
            pltpu.VMEM((_Q, 128), jnp.float32),
        ],
    )(x, x + x, cb, jnp.arange(_BN, dtype=jnp.float32).reshape(1, _BN))


_NCHUNK = 4


def kernel(x, codebook):
    n = codebook.shape[0]
    devs = [d for d in jax.devices() if d.platform == "tpu"]

    if len(devs) < 2 or n % (2 * _NCHUNK) != 0:
        _, idx = _knn_pallas(x, codebook, n)
        return idx

    chunk = n // _NCHUNK
    halfc = chunk // 2
    chunks = [codebook[c * chunk:(c + 1) * chunk] for c in range(_NCHUNK)]
    mesh = Mesh(np.array(devs[:2]), ("d",))

    def local_fn(xl, *cbls):
        shard = jax.lax.axis_index("d").astype(jnp.int32)
        best_v = best_i = None
        for c, cbl in enumerate(cbls):
            v, idx = _knn_pallas(xl, cbl, halfc)
            gidx = idx + (c * chunk + shard * halfc)
            if best_v is None:
                best_v, best_i = v, gidx
            else:
                take = (v < best_v) | ((v == best_v) & (gidx < best_i))
                best_v = jnp.where(take, v, best_v)
                best_i = jnp.where(take, gidx, best_i)
        return best_v[None], best_i[None]

    vals, idxs = _shard_map(
        local_fn, mesh=mesh,
        in_specs=(P(None, None),) + (P("d", None),) * _NCHUNK,
        out_specs=(P("d", None, None), P("d", None, None)),
        check_rep=False,
    )(x, *chunks)

    take = (vals[1] < vals[0]) | ((vals[1] == vals[0]) & (idxs[1] < idxs[0]))
    return jnp.where(take, idxs[1], idxs[0])


# speedup vs baseline: 1.2751x; 1.2751x over previous
import functools

import jax
import jax.numpy as jnp
import numpy as np
from jax.experimental import pallas as pl
from jax.experimental.pallas import tpu as pltpu

try:
    from jax.experimental.shard_map import shard_map as _shard_map
except ImportError:
    from jax import shard_map as _shard_map

from jax.sharding import Mesh, PartitionSpec as P

_Q = 1024
_D = 16
_BN = 8192


def _knn_step(x_ref, x2_ref, cb_ref, fcols_ref, oval_ref, oidx_ref,
              minval, minidx, *, n_total):
    i = pl.program_id(0)
    nsteps = pl.num_programs(0)

    x = x_ref[...]
    x2 = x2_ref[...]
    cb = cb_ref[...]

    row = jax.lax.broadcasted_iota(jnp.int32, (_BN, 1), 0)
    valid = (i * _BN + row) < n_total
    cb = jnp.where(valid, cb, 0.0)

    xsq = jnp.sum(x * x, axis=1, keepdims=True)
    csq = jnp.sum(cb * cb, axis=1, keepdims=True)
    csq = jnp.where(valid, csq, jnp.float32(3e38))

    mm2 = jax.lax.dot_general(
        x2, cb, (((1,), (1,)), ((), ())),
        preferred_element_type=jnp.float32)
    d = xsq - mm2 + csq.T

    gcols = fcols_ref[...] + (i * _BN).astype(jnp.float32)

    pairs = [(d[:, k * 128:(k + 1) * 128], gcols[:, k * 128:(k + 1) * 128])
             for k in range(_BN // 128)]
    while len(pairs) > 1:
        nxt = []
        for k in range(0, len(pairs) - 1, 2):
            (av, ai), (bv, bi) = pairs[k], pairs[k + 1]
            take_b = bv < av
            nxt.append((jnp.minimum(av, bv), jnp.where(take_b, bi, ai)))
        if len(pairs) % 2:
            nxt.append(pairs[-1])
        pairs = nxt
    lval, lid = pairs[0]

    @pl.when(i == 0)
    def _():
        minval[...] = lval
        minidx[...] = lid

    @pl.when(i > 0)
    def _():
        take = lval < minval[...]
        minval[...] = jnp.minimum(minval[...], lval)
        minidx[...] = jnp.where(take, lid, minidx[...])

    @pl.when(i == nsteps - 1)
    def _():
        accv = minval[...]
        acci = minidx[...]
        m = jnp.min(accv, axis=1, keepdims=True)
        idxf = jnp.min(
            jnp.where(accv == m, acci, jnp.float32(3e38)),
            axis=1, keepdims=True)
        oval_ref[...] = m
        oidx_ref[...] = idxf.astype(jnp.int32)


def _knn_pallas(x, cb, n_total):
    nsteps = (n_total + _BN - 1) // _BN
    return pl.pallas_call(
        functools.partial(_knn_step, n_total=n_total),
        grid=(nsteps,),
        in_specs=[
            pl.BlockSpec((_Q, _D), lambda i: (0, 0)),
            pl.BlockSpec((_Q, _D), lambda i: (0, 0)),
            pl.BlockSpec((_BN, _D), lambda i: (i, 0)),
            pl.BlockSpec((1, _BN), lambda i: (0, 0)),
        ],
        out_specs=[
            pl.BlockSpec((_Q, 1), lambda i: (0, 0)),
            pl.BlockSpec((_Q, 1), lambda i: (0, 0)),
        ],
        out_shape=[
            jax.ShapeDtypeStruct((_Q, 1), jnp.float32),
            jax.ShapeDtypeStruct((_Q, 1), jnp.int32),
        ],
        scratch_shapes=[
            pltpu.VMEM((_Q, 128), jnp.float32),
            pltpu.VMEM((_Q, 128), jnp.float32),
        ],
    )(x, x + x, cb, jnp.arange(_BN, dtype=jnp.float32).reshape(1, _BN))


def kernel(x, codebook):
    n = codebook.shape[0]
    devs = [d for d in jax.devices() if d.platform == "tpu"]

    if len(devs) < 2 or n % 2 != 0:
        _, idx = _knn_pallas(x, codebook, n)
        return idx

    half = n // 2
    mesh = Mesh(np.array(devs[:2]), ("d",))

    def local_fn(xl, cbl):
        val, idx = _knn_pallas(xl, cbl, half)
        shard = jax.lax.axis_index("d").astype(jnp.int32)
        return val[None], (idx + shard * half)[None]

    vals, idxs = _shard_map(
        local_fn, mesh=mesh,
        in_specs=(P(None, None), P("d", None)),
        out_specs=(P("d", None, None), P("d", None, None)),
        check_rep=False,
    )(x, codebook)

    take = vals[1] < vals[0]
    return jnp.where(take, idxs[1], idxs[0])
